# baseline (speedup 1.0000x reference)
import jax
import jax.numpy as jnp
from jax import lax
from jax.experimental import pallas as pl
from jax.experimental.pallas import tpu as pltpu
from jax.experimental.pallas import tpu_sc as plsc

_P = 524288
_NW = 32
_B = 128
_NPL = 24
_SPTOT = 1044480
_TOT = _SPTOT + 288000
_PPW = _P // _NW
_NCHUNK = _PPW // _B


def _sc_body(tab, crd, out, cbuf, idxb, wbuf, gbuf, obuf, gsem):
    wid = lax.axis_index("c") * 16 + lax.axis_index("s")
    iota = lax.iota(jnp.int32, 16)

    def blvl(lvl, c2):
            R = lax.shift_left(jnp.int32(64), lvl)
            Rf = R.astype(jnp.float32)
            spb = 4096 * (lax.shift_left(jnp.int32(1), 2 * lvl) - 1)
            tpb = _SPTOT + 19200 * (lax.shift_left(jnp.int32(1), lvl) - 1)
            for tpg in range(6):
                row_pl = lvl * 6 + tpg
                if tpg < 3:
                    cxi, cyi = ((0, 1), (0, 2), (1, 2))[tpg]
                    sx = 0.5 * Rf
                    ox = sx - 0.5
                    wxm = Rf - 1.0
                    sy, oy, wym = sx, ox, wxm
                    Wr = R
                    Hm1 = R - 1
                    base = spb + tpg * R * R
                else:
                    p = tpg - 3
                    cxi, cyi = 3, p
                    sx = jnp.float32(99.0)
                    ox = jnp.float32(0.0)
                    wxm = jnp.float32(99.0)
                    sy = 0.5 * (Rf - 1.0)
                    oy = sy
                    wym = Rf - 1.0
                    Wr = jnp.int32(100)
                    Hm1 = R - 1
                    base = tpb + p * 100 * R
                def gb(g, c3):
                    cx = cbuf[cxi, pl.ds(g * 16, 16)]
                    cy = cbuf[cyi, pl.ds(g * 16, 16)]
                    ix = jnp.minimum(jnp.maximum(sx * cx + ox, 0.0), wxm)
                    iy = jnp.minimum(jnp.maximum(sy * cy + oy, 0.0), wym)
                    x0 = ix.astype(jnp.int32)
                    y0 = iy.astype(jnp.int32)
                    wx = ix - x0.astype(jnp.float32)
                    wy = iy - y0.astype(jnp.float32)
                    y1 = jnp.minimum(y0 + 1, Hm1)
                    r0 = base + y0 * Wr + x0
                    r1 = base + y1 * Wr + x0
                    ioff = row_pl * 256 + g * 16
                    idxb[pl.ds(ioff, 16)] = r0
                    idxb[pl.ds(ioff + 128, 16)] = r1
                    wbuf[pl.ds(ioff, 16)] = wx
                    wbuf[pl.ds(ioff + 128, 16)] = wy
                    return c3

                lax.fori_loop(0, 8, gb, 0)
                pltpu.async_copy(
                    tab.at[idxb.at[pl.ds(row_pl * 256, _B)]],
                    gbuf.at[pl.ds(row_pl * 256, _B)],
                    gsem.at[lvl],
                )
                pltpu.async_copy(
                    tab.at[idxb.at[pl.ds(row_pl * 256 + _B, _B)]],
                    gbuf.at[pl.ds(row_pl * 256 + _B, _B)],
                    gsem.at[lvl],
                )
            return c2

    def clvl(lvl, c2):
            pltpu.make_async_copy(
                tab.at[pl.ds(0, 6 * 2 * _B)],
                gbuf.at[pl.ds(lvl * 6 * 2 * _B, 6 * 2 * _B)],
                gsem.at[lvl],
            ).wait()
            col0 = lvl * 8

            def gbody(g, c3):
                pv = iota + g * 16

                def tbody(tpg, accs):
                    row_pl = lvl * 6 + tpg
                    woff = row_pl * 256 + g * 16
                    wx = wbuf[pl.ds(woff, 16)]
                    wy = wbuf[pl.ds(woff + 128, 16)]
                    w11 = wx * wy
                    w01 = wx - w11
                    w10 = wy - w11
                    w00 = 1.0 - wx - w10
                    r0 = row_pl * 256 + pv
                    r1 = r0 + 128
                    nacc = []
                    for c in range(8):
                        c0 = jnp.full((16,), c, jnp.int32)
                        c1 = jnp.full((16,), c + 8, jnp.int32)
                        v00 = plsc.load_gather(gbuf, [r0, c0])
                        v01 = plsc.load_gather(gbuf, [r0, c1])
                        v10 = plsc.load_gather(gbuf, [r1, c0])
                        v11 = plsc.load_gather(gbuf, [r1, c1])
                        a = accs[c] + v00 * w00
                        a = a + v01 * w01
                        a = a + v10 * w10
                        nacc.append(a + v11 * w11)
                    return tuple(nacc)

                zero = jnp.zeros((16,), jnp.float32)
                accs = lax.fori_loop(0, 6, tbody, (zero,) * 8)
                for c in range(8):
                    obuf[col0 + c, pl.ds(g * 16, 16)] = accs[c]
                return c3

            lax.fori_loop(0, 8, gbody, 0)
            return c2

    pltpu.sync_copy(crd.at[:, pl.ds(wid * _PPW, _B)], cbuf)
    lax.fori_loop(0, 4, blvl, 0)

    def chunk(ci, carry):
        p0 = wid * _PPW + ci * _B
        nxt = jnp.minimum(ci + 1, _NCHUNK - 1)
        pltpu.sync_copy(crd.at[:, pl.ds(wid * _PPW + nxt * _B, _B)], cbuf)

        def lvl_step(lvl, c2):
            clvl(lvl, c2)
            blvl(lvl, c2)
            return c2

        lax.fori_loop(0, 4, lvl_step, 0)
        pltpu.sync_copy(obuf, out.at[:, pl.ds(p0, _B)])
        return carry

    lax.fori_loop(0, _NCHUNK, chunk, 0)

    def drain(lvl, c2):
        pltpu.make_async_copy(
            tab.at[pl.ds(0, 6 * 2 * _B)],
            gbuf.at[pl.ds(lvl * 6 * 2 * _B, 6 * 2 * _B)],
            gsem.at[lvl],
        ).wait()
        return c2

    lax.fori_loop(0, 4, drain, 0)


def _pack_table(sp, tp):
    parts = []
    for grid in (*sp, *tp):
        a = jnp.transpose(grid, (0, 2, 3, 1))
        a1 = jnp.concatenate([a[:, :, 1:, :], a[:, :, -1:, :]], axis=2)
        parts.append(jnp.concatenate([a, a1], axis=-1).reshape(-1, 16))
    return jnp.concatenate(parts, axis=0)


def kernel(xyz, t, sp0, sp1, sp2, sp3, tp0, tp1, tp2, tp3):
    bash = xyz.shape
    xyz = xyz.reshape(-1, xyz.shape[-1])
    t = t.reshape(-1, t.shape[-1])
    table = _pack_table((sp0, sp1, sp2, sp3), (tp0, tp1, tp2, tp3))
    coords = jnp.concatenate([xyz, t], axis=1).T

    mesh = plsc.VectorSubcoreMesh(core_axis_name="c", subcore_axis_name="s")
    f = pl.kernel(
        _sc_body,
        out_type=jax.ShapeDtypeStruct((32, _P), jnp.float32),
        mesh=mesh,
        compiler_params=pltpu.CompilerParams(
            needs_layout_passes=False, use_tc_tiling_on_sc=False),
        scratch_types=[
            pltpu.VMEM((4, _B), jnp.float32),
            pltpu.VMEM((_NPL * 2 * _B,), jnp.int32),
            pltpu.VMEM((_NPL * 2 * _B,), jnp.float32),
            pltpu.VMEM((_NPL * 2 * _B, 16), jnp.float32),
            pltpu.VMEM((32, _B), jnp.float32),
            pltpu.SemaphoreType.DMA((4,)),
        ],
    )
    val = f(table, coords).T
    return val.reshape(*bash[:-1], val.shape[-1])


# speedup vs baseline: 1.0720x; 1.0720x over previous
import jax
import jax.numpy as jnp
from jax import lax
from jax.experimental import pallas as pl
from jax.experimental.pallas import tpu as pltpu
from jax.experimental.pallas import tpu_sc as plsc

_P = 524288
_NW = 32
_B = 128
_NPL = 24
_SPTOT = 1044480
_TOT = _SPTOT + 288000
_PPW = _P // _NW
_NCHUNK = _PPW // _B


def _sc_body(tab, crd, out, cbuf, idxb, wbuf, gbuf, obuf, gsem):
    wid = lax.axis_index("c") * 16 + lax.axis_index("s")
    iota = lax.iota(jnp.int32, 16)

    def blvl(lvl, c2):
            R = lax.shift_left(jnp.int32(64), lvl)
            Rf = R.astype(jnp.float32)
            spb = 4096 * (lax.shift_left(jnp.int32(1), 2 * lvl) - 1)
            tpb = _SPTOT + 19200 * (lax.shift_left(jnp.int32(1), lvl) - 1)
            for tpg in range(6):
                row_pl = lvl * 6 + tpg
                if tpg < 3:
                    cxi, cyi = ((0, 1), (0, 2), (1, 2))[tpg]
                    sx = 0.5 * Rf
                    ox = sx - 0.5
                    wxm = Rf - 1.0
                    sy, oy, wym = sx, ox, wxm
                    Wr = R
                    Hm1 = R - 1
                    base = spb + tpg * R * R
                else:
                    p = tpg - 3
                    cxi, cyi = 3, p
                    sx = jnp.float32(99.0)
                    ox = jnp.float32(0.0)
                    wxm = jnp.float32(99.0)
                    sy = 0.5 * (Rf - 1.0)
                    oy = sy
                    wym = Rf - 1.0
                    Wr = jnp.int32(100)
                    Hm1 = R - 1
                    base = tpb + p * 100 * R
                @plsc.parallel_loop(0, 8, unroll=2)
                def gb(g):
                    cx = cbuf[cxi, pl.ds(g * 16, 16)]
                    cy = cbuf[cyi, pl.ds(g * 16, 16)]
                    ix = jnp.minimum(jnp.maximum(sx * cx + ox, 0.0), wxm)
                    iy = jnp.minimum(jnp.maximum(sy * cy + oy, 0.0), wym)
                    x0 = ix.astype(jnp.int32)
                    y0 = iy.astype(jnp.int32)
                    wx = ix - x0.astype(jnp.float32)
                    wy = iy - y0.astype(jnp.float32)
                    y1 = jnp.minimum(y0 + 1, Hm1)
                    r0 = base + y0 * Wr + x0
                    r1 = base + y1 * Wr + x0
                    ioff = row_pl * 256 + g * 16
                    idxb[pl.ds(ioff, 16)] = r0
                    idxb[pl.ds(ioff + 128, 16)] = r1
                    wbuf[pl.ds(ioff, 16)] = wx
                    wbuf[pl.ds(ioff + 128, 16)] = wy

                pltpu.async_copy(
                    tab.at[idxb.at[pl.ds(row_pl * 256, _B)]],
                    gbuf.at[pl.ds(row_pl * 256, _B)],
                    gsem.at[lvl],
                )
                pltpu.async_copy(
                    tab.at[idxb.at[pl.ds(row_pl * 256 + _B, _B)]],
                    gbuf.at[pl.ds(row_pl * 256 + _B, _B)],
                    gsem.at[lvl],
                )
            return c2

    def clvl(lvl, c2):
            pltpu.make_async_copy(
                tab.at[pl.ds(0, 6 * 2 * _B)],
                gbuf.at[pl.ds(lvl * 6 * 2 * _B, 6 * 2 * _B)],
                gsem.at[lvl],
            ).wait()
            col0 = lvl * 8

            @plsc.parallel_loop(0, 8, unroll=2)
            def gbody(g):
                pv = iota + g * 16

                def tbody(tpg, accs):
                    row_pl = lvl * 6 + tpg
                    woff = row_pl * 256 + g * 16
                    wx = wbuf[pl.ds(woff, 16)]
                    wy = wbuf[pl.ds(woff + 128, 16)]
                    w11 = wx * wy
                    w01 = wx - w11
                    w10 = wy - w11
                    w00 = 1.0 - wx - w10
                    r0 = row_pl * 256 + pv
                    r1 = r0 + 128
                    nacc = []
                    for c in range(8):
                        c0 = jnp.full((16,), c, jnp.int32)
                        c1 = jnp.full((16,), c + 8, jnp.int32)
                        v00 = plsc.load_gather(gbuf, [r0, c0])
                        v01 = plsc.load_gather(gbuf, [r0, c1])
                        v10 = plsc.load_gather(gbuf, [r1, c0])
                        v11 = plsc.load_gather(gbuf, [r1, c1])
                        a = accs[c] + v00 * w00
                        a = a + v01 * w01
                        a = a + v10 * w10
                        nacc.append(a + v11 * w11)
                    return tuple(nacc)

                zero = jnp.zeros((16,), jnp.float32)
                accs = lax.fori_loop(0, 6, tbody, (zero,) * 8)
                for c in range(8):
                    obuf[col0 + c, pl.ds(g * 16, 16)] = accs[c]

            return c2

    pltpu.sync_copy(crd.at[:, pl.ds(wid * _PPW, _B)], cbuf)
    lax.fori_loop(0, 4, blvl, 0)

    def chunk(ci, carry):
        p0 = wid * _PPW + ci * _B
        nxt = jnp.minimum(ci + 1, _NCHUNK - 1)
        pltpu.sync_copy(crd.at[:, pl.ds(wid * _PPW + nxt * _B, _B)], cbuf)

        def lvl_step(lvl, c2):
            clvl(lvl, c2)
            blvl(lvl, c2)
            return c2

        lax.fori_loop(0, 4, lvl_step, 0)
        pltpu.sync_copy(obuf, out.at[:, pl.ds(p0, _B)])
        return carry

    lax.fori_loop(0, _NCHUNK, chunk, 0)

    def drain(lvl, c2):
        pltpu.make_async_copy(
            tab.at[pl.ds(0, 6 * 2 * _B)],
            gbuf.at[pl.ds(lvl * 6 * 2 * _B, 6 * 2 * _B)],
            gsem.at[lvl],
        ).wait()
        return c2

    lax.fori_loop(0, 4, drain, 0)


def _pack_table(sp, tp):
    parts = []
    for grid in (*sp, *tp):
        a = jnp.transpose(grid, (0, 2, 3, 1))
        a1 = jnp.concatenate([a[:, :, 1:, :], a[:, :, -1:, :]], axis=2)
        parts.append(jnp.concatenate([a, a1], axis=-1).reshape(-1, 16))
    return jnp.concatenate(parts, axis=0)


def kernel(xyz, t, sp0, sp1, sp2, sp3, tp0, tp1, tp2, tp3):
    bash = xyz.shape
    xyz = xyz.reshape(-1, xyz.shape[-1])
    t = t.reshape(-1, t.shape[-1])
    table = _pack_table((sp0, sp1, sp2, sp3), (tp0, tp1, tp2, tp3))
    coords = jnp.concatenate([xyz, t], axis=1).T

    mesh = plsc.VectorSubcoreMesh(core_axis_name="c", subcore_axis_name="s")
    f = pl.kernel(
        _sc_body,
        out_type=jax.ShapeDtypeStruct((32, _P), jnp.float32),
        mesh=mesh,
        compiler_params=pltpu.CompilerParams(
            needs_layout_passes=False, use_tc_tiling_on_sc=False),
        scratch_types=[
            pltpu.VMEM((4, _B), jnp.float32),
            pltpu.VMEM((_NPL * 2 * _B,), jnp.int32),
            pltpu.VMEM((_NPL * 2 * _B,), jnp.float32),
            pltpu.VMEM((_NPL * 2 * _B, 16), jnp.float32),
            pltpu.VMEM((32, _B), jnp.float32),
            pltpu.SemaphoreType.DMA((4,)),
        ],
    )
    val = f(table, coords).T
    return val.reshape(*bash[:-1], val.shape[-1])


# speedup vs baseline: 1.0944x; 1.0209x over previous
import jax
import jax.numpy as jnp
from jax import lax
from jax.experimental import pallas as pl
from jax.experimental.pallas import tpu as pltpu
from jax.experimental.pallas import tpu_sc as plsc

_P = 524288
_NW = 32
_B = 128
_NPL = 24
_SPTOT = 1044480
_TOT = _SPTOT + 288000
_PPW = _P // _NW
_NCHUNK = _PPW // _B


def _sc_body(tab, crd, out, cbuf, idxb, wbuf, gbuf, obuf, gsem):
    wid = lax.axis_index("c") * 16 + lax.axis_index("s")
    iota = lax.iota(jnp.int32, 16)

    def blvl(lvl, c2):
            R = lax.shift_left(jnp.int32(64), lvl)
            Rf = R.astype(jnp.float32)
            spb = 4096 * (lax.shift_left(jnp.int32(1), 2 * lvl) - 1)
            tpb = _SPTOT + 19200 * (lax.shift_left(jnp.int32(1), lvl) - 1)
            for tpg in range(6):
                row_pl = lvl * 6 + tpg
                if tpg < 3:
                    cxi, cyi = ((0, 1), (0, 2), (1, 2))[tpg]
                    sx = 0.5 * Rf
                    ox = sx - 0.5
                    wxm = Rf - 1.0
                    sy, oy, wym = sx, ox, wxm
                    Wr = R
                    Hm1 = R - 1
                    base = spb + tpg * R * R
                else:
                    p = tpg - 3
                    cxi, cyi = 3, p
                    sx = jnp.float32(99.0)
                    ox = jnp.float32(0.0)
                    wxm = jnp.float32(99.0)
                    sy = 0.5 * (Rf - 1.0)
                    oy = sy
                    wym = Rf - 1.0
                    Wr = jnp.int32(100)
                    Hm1 = R - 1
                    base = tpb + p * 100 * R
                @plsc.parallel_loop(0, 8, unroll=2)
                def gb(g):
                    cx = cbuf[cxi, pl.ds(g * 16, 16)]
                    cy = cbuf[cyi, pl.ds(g * 16, 16)]
                    ix = jnp.minimum(jnp.maximum(sx * cx + ox, 0.0), wxm)
                    iy = jnp.minimum(jnp.maximum(sy * cy + oy, 0.0), wym)
                    x0 = ix.astype(jnp.int32)
                    y0 = iy.astype(jnp.int32)
                    wx = ix - x0.astype(jnp.float32)
                    wy = iy - y0.astype(jnp.float32)
                    y1 = jnp.minimum(y0 + 1, Hm1)
                    r0 = base + y0 * Wr + x0
                    r1 = base + y1 * Wr + x0
                    ioff = row_pl * 256 + g * 16
                    idxb[pl.ds(ioff, 16)] = r0
                    idxb[pl.ds(ioff + 128, 16)] = r1
                    wbuf[pl.ds(ioff, 16)] = wx
                    wbuf[pl.ds(ioff + 128, 16)] = wy

                pltpu.async_copy(
                    tab.at[idxb.at[pl.ds(row_pl * 256, _B)]],
                    gbuf.at[pl.ds(row_pl * 256, _B)],
                    gsem.at[lvl],
                )
                pltpu.async_copy(
                    tab.at[idxb.at[pl.ds(row_pl * 256 + _B, _B)]],
                    gbuf.at[pl.ds(row_pl * 256 + _B, _B)],
                    gsem.at[lvl],
                )
            return c2

    def clvl(lvl, c2):
            pltpu.make_async_copy(
                tab.at[pl.ds(0, 6 * 2 * _B)],
                gbuf.at[pl.ds(lvl * 6 * 2 * _B, 6 * 2 * _B)],
                gsem.at[lvl],
            ).wait()
            col0 = lvl * 8

            @plsc.parallel_loop(0, 8, unroll=4)
            def gbody(g):
                pv = iota + g * 16

                def tbody(tpg, accs):
                    row_pl = lvl * 6 + tpg
                    woff = row_pl * 256 + g * 16
                    wx = wbuf[pl.ds(woff, 16)]
                    wy = wbuf[pl.ds(woff + 128, 16)]
                    w11 = wx * wy
                    w01 = wx - w11
                    w10 = wy - w11
                    w00 = 1.0 - wx - w10
                    r0 = row_pl * 256 + pv
                    r1 = r0 + 128
                    nacc = []
                    for c in range(8):
                        c0 = jnp.full((16,), c, jnp.int32)
                        c1 = jnp.full((16,), c + 8, jnp.int32)
                        v00 = plsc.load_gather(gbuf, [r0, c0])
                        v01 = plsc.load_gather(gbuf, [r0, c1])
                        v10 = plsc.load_gather(gbuf, [r1, c0])
                        v11 = plsc.load_gather(gbuf, [r1, c1])
                        a = accs[c] + v00 * w00
                        a = a + v01 * w01
                        a = a + v10 * w10
                        nacc.append(a + v11 * w11)
                    return tuple(nacc)

                zero = jnp.zeros((16,), jnp.float32)
                accs = lax.fori_loop(0, 6, tbody, (zero,) * 8)
                for c in range(8):
                    obuf[col0 + c, pl.ds(g * 16, 16)] = accs[c]

            return c2

    pltpu.sync_copy(crd.at[:, pl.ds(wid * _PPW, _B)], cbuf)
    lax.fori_loop(0, 4, blvl, 0)

    def chunk(ci, carry):
        p0 = wid * _PPW + ci * _B
        nxt = jnp.minimum(ci + 1, _NCHUNK - 1)
        pltpu.sync_copy(crd.at[:, pl.ds(wid * _PPW + nxt * _B, _B)], cbuf)

        def lvl_step(lvl, c2):
            clvl(lvl, c2)
            blvl(lvl, c2)
            return c2

        lax.fori_loop(0, 4, lvl_step, 0)
        pltpu.sync_copy(obuf, out.at[:, pl.ds(p0, _B)])
        return carry

    lax.fori_loop(0, _NCHUNK, chunk, 0)

    def drain(lvl, c2):
        pltpu.make_async_copy(
            tab.at[pl.ds(0, 6 * 2 * _B)],
            gbuf.at[pl.ds(lvl * 6 * 2 * _B, 6 * 2 * _B)],
            gsem.at[lvl],
        ).wait()
        return c2

    lax.fori_loop(0, 4, drain, 0)


def _pack_table(sp, tp):
    parts = []
    for grid in (*sp, *tp):
        a = jnp.transpose(grid, (0, 2, 3, 1))
        a1 = jnp.concatenate([a[:, :, 1:, :], a[:, :, -1:, :]], axis=2)
        parts.append(jnp.concatenate([a, a1], axis=-1).reshape(-1, 16))
    return jnp.concatenate(parts, axis=0)


def kernel(xyz, t, sp0, sp1, sp2, sp3, tp0, tp1, tp2, tp3):
    bash = xyz.shape
    xyz = xyz.reshape(-1, xyz.shape[-1])
    t = t.reshape(-1, t.shape[-1])
    table = _pack_table((sp0, sp1, sp2, sp3), (tp0, tp1, tp2, tp3))
    coords = jnp.concatenate([xyz, t], axis=1).T

    mesh = plsc.VectorSubcoreMesh(core_axis_name="c", subcore_axis_name="s")
    f = pl.kernel(
        _sc_body,
        out_type=jax.ShapeDtypeStruct((32, _P), jnp.float32),
        mesh=mesh,
        compiler_params=pltpu.CompilerParams(
            needs_layout_passes=False, use_tc_tiling_on_sc=False),
        scratch_types=[
            pltpu.VMEM((4, _B), jnp.float32),
            pltpu.VMEM((_NPL * 2 * _B,), jnp.int32),
            pltpu.VMEM((_NPL * 2 * _B,), jnp.float32),
            pltpu.VMEM((_NPL * 2 * _B, 16), jnp.float32),
            pltpu.VMEM((32, _B), jnp.float32),
            pltpu.SemaphoreType.DMA((4,)),
        ],
    )
    val = f(table, coords).T
    return val.reshape(*bash[:-1], val.shape[-1])
